# Initial kernel scaffold; baseline (speedup 1.0000x reference)
import jax
import jax.numpy as jnp
from jax.experimental import pallas as pl


def kernel(feat, edge_index_12, edge_index_23, edge_index_34, W, b):
    raise NotImplementedError("write your pallas kernel here")


# speedup vs baseline: 2.5786x; 2.5786x over previous
import functools

import jax
import jax.numpy as jnp
from jax import lax
from jax.experimental import pallas as pl
from jax.experimental.pallas import tpu as pltpu
from jax.experimental.pallas import tpu_sc as plsc

NEG_BIG = -1e30
CW = 32
NCHUNK = 4
B = 128
NTILES = 16
F32 = jnp.float32
I32 = jnp.int32

N1, N2, N3, N4 = 50000, 25000, 10000, 2000
A1, A2, A3, A4 = 51200, 26624, 12288, 4096

@functools.cache
def _mesh():
    return plsc.VectorSubcoreMesh(core_axis_name="core", subcore_axis_name="subcore")


def _pad_to(x, n, fill):
    return jnp.concatenate([x, jnp.full((n - x.shape[0],), fill, x.dtype)])


def _zero_fill(zbuf):
    zeros = jnp.zeros((16,), F32)

    @pl.loop(0, zbuf.shape[0])
    def _(r):
        for v in range(CW // 16):
            zbuf[r, pl.ds(16 * v, 16)] = zeros


def _zero_stripe(zbuf, acc, stripe, s):
    @pl.loop(0, stripe, step=B)
    def _(r0):
        pltpu.sync_copy(zbuf, acc.at[pl.ds(s * stripe + r0, B)])


def _accumulate(gidx, sidx, nb, s, tabs, rows, accs, gbuf, sbuf):
    base = s * nb * B

    @pl.loop(0, nb)
    def _(bi):
        off = base + bi * B
        pltpu.sync_copy(gidx.at[pl.ds(off, B)], gbuf)
        pltpu.sync_copy(sidx.at[pl.ds(off, B)], sbuf)
        for tab, row in zip(tabs, rows):
            pltpu.sync_copy(tab.at[gbuf], row)
        for row, acc in zip(rows, accs):
            pltpu.sync_copy(row, acc.at[sbuf], add=True)


def _parity_to_sign(buf, r):
    out = []
    for v in range(CW // 16):
        cnt = buf[r, pl.ds(16 * v, 16)].astype(I32)
        par = (cnt & 1).astype(F32)
        out.append(1.0 - 2.0 * par)
    return out


def _make_up_phase(e_pad, n_acc, last):
    nb = e_pad // (NTILES * B)
    stripe = n_acc // NTILES
    if last:
        out_type = [jax.ShapeDtypeStruct((n_acc, CW), F32) for _ in range(NCHUNK)]
    else:
        out_type = [jax.ShapeDtypeStruct((n_acc, CW), F32) for _ in range(2 * NCHUNK)]

    @functools.partial(
        pl.kernel,
        mesh=_mesh(),
        out_type=out_type,
        compiler_params=pltpu.CompilerParams(use_tc_tiling_on_sc=False),
        scratch_types=[
            pltpu.VMEM((B,), I32),
            pltpu.VMEM((B,), I32),
            pltpu.VMEM((B, CW), F32),
            pltpu.VMEM((B, CW), F32),
            pltpu.VMEM((B, CW), F32),
            pltpu.VMEM((B, CW), F32),
            pltpu.VMEM((B, CW), F32),
            pltpu.VMEM_SHARED((n_acc, CW), F32),
            pltpu.VMEM_SHARED((n_acc, CW), F32),
        ],
    )
    def up_kernel(cidx, pidx, *refs):
        ltabs = refs[:NCHUNK]
        stabs = refs[NCHUNK:2 * NCHUNK]
        outs = refs[2 * NCHUNK:2 * NCHUNK + len(out_type)]
        (gbuf, sbuf, rowL, rowS, zbuf, dbufL, dbufS, accL, accS) = refs[
            2 * NCHUNK + len(out_type):]
        core = lax.axis_index("core")
        s = lax.axis_index("subcore")
        _zero_fill(zbuf)

        def run_pass(chunk):
            _zero_stripe(zbuf, accL, stripe, s)
            _zero_stripe(zbuf, accS, stripe, s)
            plsc.subcore_barrier()
            _accumulate(cidx, pidx, nb, s, (ltabs[chunk], stabs[chunk]),
                        (rowL, rowS), (accL, accS), gbuf, sbuf)
            plsc.subcore_barrier()

            @pl.loop(0, stripe, step=B)
            def _(r0):
                row = s * stripe + r0
                pltpu.sync_copy(accL.at[pl.ds(row, B)], dbufL)
                pltpu.sync_copy(accS.at[pl.ds(row, B)], dbufS)

                @pl.loop(0, B)
                def _(r):
                    sgn = _parity_to_sign(dbufS, r)
                    for v in range(CW // 16):
                        sl = pl.ds(16 * v, 16)
                        if last:
                            dbufL[r, sl] = sgn[v] * jnp.exp(dbufL[r, sl])
                        else:
                            dbufS[r, sl] = 0.5 * (1.0 - sgn[v])

                if last:
                    pltpu.sync_copy(dbufL, outs[chunk].at[pl.ds(row, B)])
                else:
                    pltpu.sync_copy(dbufL, outs[chunk].at[pl.ds(row, B)])
                    pltpu.sync_copy(dbufS, outs[NCHUNK + chunk].at[pl.ds(row, B)])

            plsc.subcore_barrier()

        for ci in range(2):
            @pl.when(core == ci)
            def _():
                for p in range(2):
                    run_pass(2 * ci + p)

    return up_kernel


def _make_down_phase(e_pad, n_acc, materialize):
    nb = e_pad // (NTILES * B)
    stripe = n_acc // NTILES
    n_in = 3 * NCHUNK if materialize else NCHUNK
    out_type = [jax.ShapeDtypeStruct((n_acc, CW), F32) for _ in range(NCHUNK)]

    @functools.partial(
        pl.kernel,
        mesh=_mesh(),
        out_type=out_type,
        compiler_params=pltpu.CompilerParams(use_tc_tiling_on_sc=False),
        scratch_types=[
            pltpu.VMEM((B,), I32),
            pltpu.VMEM((B,), I32),
            pltpu.VMEM((B, CW), F32),
            pltpu.VMEM((B, CW), F32),
            pltpu.VMEM((B, CW), F32),
            pltpu.VMEM((B, CW), F32),
            pltpu.VMEM((B, CW), F32),
            pltpu.VMEM_SHARED((n_acc, CW), F32),
        ],
    )
    def down_kernel(pidx, cidx, *refs):
        htabs = refs[:NCHUNK]
        if materialize:
            ltabs = refs[NCHUNK:2 * NCHUNK]
            stabs = refs[2 * NCHUNK:3 * NCHUNK]
        outs = refs[n_in:n_in + NCHUNK]
        (gbuf, sbuf, rowH, zbuf, dbufD, dbufL, dbufS, accD) = refs[n_in + NCHUNK:]
        core = lax.axis_index("core")
        s = lax.axis_index("subcore")
        _zero_fill(zbuf)

        def run_pass(chunk):
            _zero_stripe(zbuf, accD, stripe, s)
            plsc.subcore_barrier()
            _accumulate(pidx, cidx, nb, s, (htabs[chunk],), (rowH,),
                        (accD,), gbuf, sbuf)
            plsc.subcore_barrier()

            @pl.loop(0, stripe, step=B)
            def _(r0):
                row = s * stripe + r0
                pltpu.sync_copy(accD.at[pl.ds(row, B)], dbufD)
                if materialize:
                    pltpu.sync_copy(ltabs[chunk].at[pl.ds(row, B)], dbufL)
                    pltpu.sync_copy(stabs[chunk].at[pl.ds(row, B)], dbufS)

                    @pl.loop(0, B)
                    def _(r):
                        sgn = _parity_to_sign(dbufS, r)
                        for v in range(CW // 16):
                            sl = pl.ds(16 * v, 16)
                            dbufD[r, sl] = (
                                sgn[v] * jnp.exp(dbufL[r, sl]) + dbufD[r, sl])

                pltpu.sync_copy(dbufD, outs[chunk].at[pl.ds(row, B)])

            plsc.subcore_barrier()

        for ci in range(2):
            @pl.when(core == ci)
            def _():
                for p in range(2):
                    run_pass(2 * ci + p)

    return down_kernel


def _tc_prep(feat):
    rb = 200
    grid = N2 // rb

    def body(feat_ref, *outs):
        x = feat_ref[...]
        logx = jnp.where(x == 0.0, F32(NEG_BIG), jnp.log(jnp.abs(x)))
        sgn = (x < 0.0).astype(F32)
        for c in range(NCHUNK):
            outs[c][...] = logx[:, 32 * c:32 * c + 32]
            outs[NCHUNK + c][...] = sgn[:, 32 * c:32 * c + 32]

    return pl.pallas_call(
        body,
        grid=(grid,),
        in_specs=[pl.BlockSpec((rb, 128), lambda i: (i, 0))],
        out_specs=[pl.BlockSpec((rb, CW), lambda i: (i, 0))] * (2 * NCHUNK),
        out_shape=[jax.ShapeDtypeStruct((N2, CW), F32)] * (2 * NCHUNK),
    )(feat)


def _tc_final(feat, d1c, W, b):
    rb = 400
    grid = N1 // rb

    def body(feat_ref, d0, d1, d2, d3, w_ref, b_ref, out_ref):
        d = jnp.concatenate([d0[...], d1[...], d2[...], d3[...]], axis=1)
        h = feat_ref[...] + d
        y = jax.lax.dot_general(h, w_ref[...], (((1,), (0,)), ((), ())),
                                preferred_element_type=F32,
                                precision=jax.lax.Precision.HIGHEST)
        y = y + b_ref[...]
        out_ref[...] = y * (1.0 / (1.0 + jnp.exp(-y)))

    return pl.pallas_call(
        body,
        grid=(grid,),
        in_specs=[pl.BlockSpec((rb, 128), lambda i: (i, 0))]
        + [pl.BlockSpec((rb, CW), lambda i: (i, 0))] * NCHUNK
        + [pl.BlockSpec((128, 128), lambda i: (0, 0)),
           pl.BlockSpec((128,), lambda i: (0,))],
        out_specs=pl.BlockSpec((rb, 128), lambda i: (i, 0)),
        out_shape=jax.ShapeDtypeStruct((N1, 128), F32),
    )(feat, *d1c, W, b)


def kernel(feat, edge_index_12, edge_index_23, edge_index_34, W, b):
    e12, e23, e34 = 501760, 102400, 20480

    c12u = _pad_to(edge_index_12[0], e12, 0)
    p12u = _pad_to(edge_index_12[1], e12, N2)
    c23u = _pad_to(edge_index_23[0], e23, 0)
    p23u = _pad_to(edge_index_23[1], e23, N3)
    c34u = _pad_to(edge_index_34[0], e34, 0)
    p34u = _pad_to(edge_index_34[1], e34, N4)
    p34d = _pad_to(edge_index_34[1], e34, 0)
    c34d = _pad_to(edge_index_34[0], e34, N3)
    p23d = _pad_to(edge_index_23[1], e23, 0)
    c23d = _pad_to(edge_index_23[0], e23, N2)
    p12d = _pad_to(edge_index_12[1], e12, 0)
    c12d = _pad_to(edge_index_12[0], e12, N1)

    tabs1 = _tc_prep(feat)

    up12 = _make_up_phase(e12, A2, last=False)
    tabs2 = up12(c12u, p12u, *tabs1)
    up23 = _make_up_phase(e23, A3, last=False)
    tabs3 = up23(c23u, p23u, *tabs2)
    up34 = _make_up_phase(e34, A4, last=True)
    h4 = up34(c34u, p34u, *tabs3)

    down34 = _make_down_phase(e34, A3, materialize=True)
    h3 = down34(p34d, c34d, *h4, *tabs3)
    down23 = _make_down_phase(e23, A2, materialize=True)
    h2 = down23(p23d, c23d, *h3, *tabs2)
    down12 = _make_down_phase(e12, A1, materialize=False)
    d1 = down12(p12d, c12d, *h2)

    return _tc_final(feat, d1, W, b)


# speedup vs baseline: 3.5234x; 1.3664x over previous
import functools

import jax
import jax.numpy as jnp
from jax import lax
from jax.experimental import pallas as pl
from jax.experimental.pallas import tpu as pltpu
from jax.experimental.pallas import tpu_sc as plsc

NEG_BIG = -1e30
CW = 32
NCHUNK = 4
B = 128
NTILES = 16
F32 = jnp.float32
I32 = jnp.int32

N1, N2, N3, N4 = 50000, 25000, 10000, 2000
A1, A2, A3, A4 = 51200, 26624, 12288, 4096

@functools.cache
def _mesh():
    return plsc.VectorSubcoreMesh(core_axis_name="core", subcore_axis_name="subcore")


def _pad_to(x, n, fill):
    pad = jnp.concatenate([x, jnp.full((n - x.shape[0],), fill, x.dtype)])
    return pad.reshape(n // B, B)


def _zero_fill(zbuf):
    zeros = jnp.zeros((16,), F32)

    @pl.loop(0, zbuf.shape[0])
    def _(r):
        for v in range(CW // 16):
            zbuf[r, pl.ds(16 * v, 16)] = zeros


def _zero_stripe(zbuf, acc, stripe, s):
    @pl.loop(0, stripe, step=B)
    def _(r0):
        pltpu.sync_copy(zbuf, acc.at[pl.ds(s * stripe + r0, B)])


SB = 4


def _accumulate(gidx, sidx, nb, s, tabs, rowslots, accs, gsb, ssb, sems):
    @pl.loop(0, nb // SB)
    def _(sbi):
        r0 = s * nb + sbi * SB
        pltpu.sync_copy(gidx.at[pl.ds(r0, SB)], gsb)
        pltpu.sync_copy(sidx.at[pl.ds(r0, SB)], ssb)
        descs = {}

        def issue(j):
            sl = j & 1
            for t, (tab, slots, sem2) in enumerate(zip(tabs, rowslots, sems)):
                descs[(j, t)] = pltpu.async_copy(
                    tab.at[gsb.at[j]], slots[sl], sem2[sl])

        issue(0)
        for j in range(SB):
            sl = j & 1
            if j + 1 < SB:
                issue(j + 1)
            for t in range(len(tabs)):
                descs[(j, t)].wait()
            for slots, acc in zip(rowslots, accs):
                pltpu.sync_copy(slots[sl], acc.at[ssb.at[j]], add=True)


def _parity_to_sign(buf, r):
    out = []
    for v in range(CW // 16):
        cnt = buf[r, pl.ds(16 * v, 16)].astype(I32)
        par = (cnt & 1).astype(F32)
        out.append(1.0 - 2.0 * par)
    return out


def _make_up_phase(e_pad, n_acc, last):
    nb = e_pad // (NTILES * B)
    stripe = n_acc // NTILES
    if last:
        out_type = [jax.ShapeDtypeStruct((n_acc, CW), F32) for _ in range(NCHUNK)]
    else:
        out_type = [jax.ShapeDtypeStruct((n_acc, CW), F32) for _ in range(2 * NCHUNK)]

    @functools.partial(
        pl.kernel,
        mesh=_mesh(),
        out_type=out_type,
        compiler_params=pltpu.CompilerParams(use_tc_tiling_on_sc=False),
        scratch_types=[
            pltpu.VMEM((SB, B), I32),
            pltpu.VMEM((SB, B), I32),
            pltpu.VMEM((B, CW), F32),
            pltpu.VMEM((B, CW), F32),
            pltpu.VMEM((B, CW), F32),
            pltpu.VMEM((B, CW), F32),
            pltpu.VMEM_SHARED((n_acc, CW), F32),
            pltpu.VMEM_SHARED((n_acc, CW), F32),
            pltpu.SemaphoreType.DMA,
            pltpu.SemaphoreType.DMA,
        ],
    )
    def up_kernel(cidx, pidx, *refs):
        ltabs = refs[:NCHUNK]
        stabs = refs[NCHUNK:2 * NCHUNK]
        outs = refs[2 * NCHUNK:2 * NCHUNK + len(out_type)]
        (gsb, ssb, rowLa, rowLb, rowSa, rowSb,
         accL, accS, semA, semB) = refs[
            2 * NCHUNK + len(out_type):]
        core = lax.axis_index("core")
        s = lax.axis_index("subcore")
        dbufL, dbufS = rowLa, rowSa

        def run_pass(chunk):
            _zero_fill(rowLb)
            _zero_stripe(rowLb, accL, stripe, s)
            _zero_stripe(rowLb, accS, stripe, s)
            plsc.subcore_barrier()
            _accumulate(cidx, pidx, nb, s, (ltabs[chunk], stabs[chunk]),
                        ((rowLa, rowLb), (rowSa, rowSb)), (accL, accS),
                        gsb, ssb, ((semA, semB), (semA, semB)))
            plsc.subcore_barrier()

            @pl.loop(0, stripe, step=B)
            def _(r0):
                row = s * stripe + r0
                pltpu.sync_copy(accL.at[pl.ds(row, B)], dbufL)
                pltpu.sync_copy(accS.at[pl.ds(row, B)], dbufS)

                @pl.loop(0, B)
                def _(r):
                    sgn = _parity_to_sign(dbufS, r)
                    for v in range(CW // 16):
                        sl = pl.ds(16 * v, 16)
                        if last:
                            dbufL[r, sl] = sgn[v] * jnp.exp(dbufL[r, sl])
                        else:
                            dbufS[r, sl] = 0.5 * (1.0 - sgn[v])

                if last:
                    pltpu.sync_copy(dbufL, outs[chunk].at[pl.ds(row, B)])
                else:
                    pltpu.sync_copy(dbufL, outs[chunk].at[pl.ds(row, B)])
                    pltpu.sync_copy(dbufS, outs[NCHUNK + chunk].at[pl.ds(row, B)])

            plsc.subcore_barrier()

        for ci in range(2):
            @pl.when(core == ci)
            def _():
                for p in range(2):
                    run_pass(2 * ci + p)

    return up_kernel


def _make_down_phase(e_pad, n_acc, materialize):
    nb = e_pad // (NTILES * B)
    stripe = n_acc // NTILES
    n_in = 3 * NCHUNK if materialize else NCHUNK
    out_type = [jax.ShapeDtypeStruct((n_acc, CW), F32) for _ in range(NCHUNK)]

    @functools.partial(
        pl.kernel,
        mesh=_mesh(),
        out_type=out_type,
        compiler_params=pltpu.CompilerParams(use_tc_tiling_on_sc=False),
        scratch_types=[
            pltpu.VMEM((SB, B), I32),
            pltpu.VMEM((SB, B), I32),
            pltpu.VMEM((B, CW), F32),
            pltpu.VMEM((B, CW), F32),
            pltpu.VMEM((B, CW), F32),
            pltpu.VMEM_SHARED((n_acc, CW), F32),
            pltpu.SemaphoreType.DMA,
            pltpu.SemaphoreType.DMA,
        ],
    )
    def down_kernel(pidx, cidx, *refs):
        htabs = refs[:NCHUNK]
        if materialize:
            ltabs = refs[NCHUNK:2 * NCHUNK]
            stabs = refs[2 * NCHUNK:3 * NCHUNK]
        outs = refs[n_in:n_in + NCHUNK]
        (gsb, ssb, rowHa, rowHb, xbuf, accD,
         semHa, semHb) = refs[n_in + NCHUNK:]
        core = lax.axis_index("core")
        s = lax.axis_index("subcore")
        dbufD, dbufL, dbufS = rowHa, rowHb, xbuf

        def run_pass(chunk):
            _zero_fill(rowHb)
            _zero_stripe(rowHb, accD, stripe, s)
            plsc.subcore_barrier()
            _accumulate(pidx, cidx, nb, s, (htabs[chunk],), ((rowHa, rowHb),),
                        (accD,), gsb, ssb, ((semHa, semHb),))
            plsc.subcore_barrier()

            @pl.loop(0, stripe, step=B)
            def _(r0):
                row = s * stripe + r0
                pltpu.sync_copy(accD.at[pl.ds(row, B)], dbufD)
                if materialize:
                    pltpu.sync_copy(ltabs[chunk].at[pl.ds(row, B)], dbufL)
                    pltpu.sync_copy(stabs[chunk].at[pl.ds(row, B)], dbufS)

                    @pl.loop(0, B)
                    def _(r):
                        sgn = _parity_to_sign(dbufS, r)
                        for v in range(CW // 16):
                            sl = pl.ds(16 * v, 16)
                            dbufD[r, sl] = (
                                sgn[v] * jnp.exp(dbufL[r, sl]) + dbufD[r, sl])

                pltpu.sync_copy(dbufD, outs[chunk].at[pl.ds(row, B)])

            plsc.subcore_barrier()

        for ci in range(2):
            @pl.when(core == ci)
            def _():
                for p in range(2):
                    run_pass(2 * ci + p)

    return down_kernel


def _tc_prep(feat):
    rb = 200
    grid = N2 // rb

    def body(feat_ref, *outs):
        x = feat_ref[...]
        logx = jnp.where(x == 0.0, F32(NEG_BIG), jnp.log(jnp.abs(x)))
        sgn = (x < 0.0).astype(F32)
        for c in range(NCHUNK):
            outs[c][...] = logx[:, 32 * c:32 * c + 32]
            outs[NCHUNK + c][...] = sgn[:, 32 * c:32 * c + 32]

    return pl.pallas_call(
        body,
        grid=(grid,),
        in_specs=[pl.BlockSpec((rb, 128), lambda i: (i, 0))],
        out_specs=[pl.BlockSpec((rb, CW), lambda i: (i, 0))] * (2 * NCHUNK),
        out_shape=[jax.ShapeDtypeStruct((N2, CW), F32)] * (2 * NCHUNK),
    )(feat)


def _tc_final(feat, d1c, W, b):
    rb = 400
    grid = N1 // rb

    def body(feat_ref, d0, d1, d2, d3, w_ref, b_ref, out_ref):
        d = jnp.concatenate([d0[...], d1[...], d2[...], d3[...]], axis=1)
        h = feat_ref[...] + d
        y = jax.lax.dot_general(h, w_ref[...], (((1,), (0,)), ((), ())),
                                preferred_element_type=F32,
                                precision=jax.lax.Precision.HIGHEST)
        y = y + b_ref[...]
        out_ref[...] = y * (1.0 / (1.0 + jnp.exp(-y)))

    return pl.pallas_call(
        body,
        grid=(grid,),
        in_specs=[pl.BlockSpec((rb, 128), lambda i: (i, 0))]
        + [pl.BlockSpec((rb, CW), lambda i: (i, 0))] * NCHUNK
        + [pl.BlockSpec((128, 128), lambda i: (0, 0)),
           pl.BlockSpec((128,), lambda i: (0,))],
        out_specs=pl.BlockSpec((rb, 128), lambda i: (i, 0)),
        out_shape=jax.ShapeDtypeStruct((N1, 128), F32),
    )(feat, *d1c, W, b)


def kernel(feat, edge_index_12, edge_index_23, edge_index_34, W, b):
    e12, e23, e34 = 507904, 106496, 24576

    c12u = _pad_to(edge_index_12[0], e12, 0)
    p12u = _pad_to(edge_index_12[1], e12, N2)
    c23u = _pad_to(edge_index_23[0], e23, 0)
    p23u = _pad_to(edge_index_23[1], e23, N3)
    c34u = _pad_to(edge_index_34[0], e34, 0)
    p34u = _pad_to(edge_index_34[1], e34, N4)
    p34d = _pad_to(edge_index_34[1], e34, 0)
    c34d = _pad_to(edge_index_34[0], e34, N3)
    p23d = _pad_to(edge_index_23[1], e23, 0)
    c23d = _pad_to(edge_index_23[0], e23, N2)
    p12d = _pad_to(edge_index_12[1], e12, 0)
    c12d = _pad_to(edge_index_12[0], e12, N1)

    tabs1 = _tc_prep(feat)

    up12 = _make_up_phase(e12, A2, last=False)
    tabs2 = up12(c12u, p12u, *tabs1)
    up23 = _make_up_phase(e23, A3, last=False)
    tabs3 = up23(c23u, p23u, *tabs2)
    up34 = _make_up_phase(e34, A4, last=True)
    h4 = up34(c34u, p34u, *tabs3)

    down34 = _make_down_phase(e34, A3, materialize=True)
    h3 = down34(p34d, c34d, *h4, *tabs3)
    down23 = _make_down_phase(e23, A2, materialize=True)
    h2 = down23(p23d, c23d, *h3, *tabs2)
    down12 = _make_down_phase(e12, A1, materialize=False)
    d1 = down12(p12d, c12d, *h2)

    return _tc_final(feat, d1, W, b)


# speedup vs baseline: 3.5619x; 1.0109x over previous
import functools

import jax
import jax.numpy as jnp
from jax import lax
from jax.experimental import pallas as pl
from jax.experimental.pallas import tpu as pltpu
from jax.experimental.pallas import tpu_sc as plsc

NEG_BIG = -1e30
CW = 32
NCHUNK = 4
B = 128
NTILES = 16
F32 = jnp.float32
I32 = jnp.int32

N1, N2, N3, N4 = 50000, 25000, 10000, 2000
A1, A2, A3, A4 = 51200, 26624, 12288, 4096

@functools.cache
def _mesh():
    return plsc.VectorSubcoreMesh(core_axis_name="core", subcore_axis_name="subcore")


def _pad_to(x, n, fill):
    pad = jnp.concatenate([x, jnp.full((n - x.shape[0],), fill, x.dtype)])
    return pad.reshape(n // B, B)


def _zero_fill(zbuf):
    zeros = jnp.zeros((16,), F32)

    @pl.loop(0, zbuf.shape[0])
    def _(r):
        for v in range(CW // 16):
            zbuf[r, pl.ds(16 * v, 16)] = zeros


def _zero_stripe(zbuf, acc, stripe, s):
    @pl.loop(0, stripe, step=B)
    def _(r0):
        pltpu.sync_copy(zbuf, acc.at[pl.ds(s * stripe + r0, B)])


SB = 4


def _accumulate(gidx, sidx, nb, s, tabs, rowslots, accs, gsb, ssb, sems, scsems):
    @pl.loop(0, nb // SB)
    def _(sbi):
        r0 = s * nb + sbi * SB
        pltpu.sync_copy(gidx.at[pl.ds(r0, SB)], gsb)
        pltpu.sync_copy(sidx.at[pl.ds(r0, SB)], ssb)
        descs, scds = {}, {}

        def issue(j):
            sl = j & 1
            for t, (tab, slots, sem2) in enumerate(zip(tabs, rowslots, sems)):
                descs[(j, t)] = pltpu.async_copy(
                    tab.at[gsb.at[j]], slots[sl], sem2[sl])

        issue(0)
        for j in range(SB):
            sl = j & 1
            if j + 1 < SB:
                for t in range(len(tabs)):
                    if (j - 1, t) in scds:
                        scds[(j - 1, t)].wait()
                issue(j + 1)
            for t in range(len(tabs)):
                descs[(j, t)].wait()
            for t, (slots, acc) in enumerate(zip(rowslots, accs)):
                scds[(j, t)] = pltpu.async_copy(
                    slots[sl], acc.at[ssb.at[j]], scsems[sl], add=True)
        for j in (SB - 2, SB - 1):
            for t in range(len(tabs)):
                scds[(j, t)].wait()


def _parity_to_sign(buf, r):
    out = []
    for v in range(CW // 16):
        cnt = buf[r, pl.ds(16 * v, 16)].astype(I32)
        par = (cnt & 1).astype(F32)
        out.append(1.0 - 2.0 * par)
    return out


def _make_up_phase(e_pad, n_acc, last):
    nb = e_pad // (NTILES * B)
    stripe = n_acc // NTILES
    if last:
        out_type = [jax.ShapeDtypeStruct((n_acc, CW), F32) for _ in range(NCHUNK)]
    else:
        out_type = [jax.ShapeDtypeStruct((n_acc, CW), F32) for _ in range(2 * NCHUNK)]

    @functools.partial(
        pl.kernel,
        mesh=_mesh(),
        out_type=out_type,
        compiler_params=pltpu.CompilerParams(use_tc_tiling_on_sc=False),
        scratch_types=[
            pltpu.VMEM((SB, B), I32),
            pltpu.VMEM((SB, B), I32),
            pltpu.VMEM((B, CW), F32),
            pltpu.VMEM((B, CW), F32),
            pltpu.VMEM((B, CW), F32),
            pltpu.VMEM((B, CW), F32),
            pltpu.VMEM_SHARED((n_acc, CW), F32),
            pltpu.VMEM_SHARED((n_acc, CW), F32),
            pltpu.SemaphoreType.DMA,
            pltpu.SemaphoreType.DMA,
            pltpu.SemaphoreType.DMA,
            pltpu.SemaphoreType.DMA,
        ],
    )
    def up_kernel(cidx, pidx, *refs):
        ltabs = refs[:NCHUNK]
        stabs = refs[NCHUNK:2 * NCHUNK]
        outs = refs[2 * NCHUNK:2 * NCHUNK + len(out_type)]
        (gsb, ssb, rowLa, rowLb, rowSa, rowSb,
         accL, accS, semA, semB, semC, semD) = refs[
            2 * NCHUNK + len(out_type):]
        core = lax.axis_index("core")
        s = lax.axis_index("subcore")
        dbufL, dbufS = rowLa, rowSa

        def run_pass(chunk):
            _zero_fill(rowLb)
            _zero_stripe(rowLb, accL, stripe, s)
            _zero_stripe(rowLb, accS, stripe, s)
            plsc.subcore_barrier()
            _accumulate(cidx, pidx, nb, s, (ltabs[chunk], stabs[chunk]),
                        ((rowLa, rowLb), (rowSa, rowSb)), (accL, accS),
                        gsb, ssb, ((semA, semB), (semA, semB)), (semC, semD))
            plsc.subcore_barrier()

            @pl.loop(0, stripe, step=B)
            def _(r0):
                row = s * stripe + r0
                pltpu.sync_copy(accL.at[pl.ds(row, B)], dbufL)
                pltpu.sync_copy(accS.at[pl.ds(row, B)], dbufS)

                @pl.loop(0, B)
                def _(r):
                    sgn = _parity_to_sign(dbufS, r)
                    for v in range(CW // 16):
                        sl = pl.ds(16 * v, 16)
                        if last:
                            dbufL[r, sl] = sgn[v] * jnp.exp(dbufL[r, sl])
                        else:
                            dbufS[r, sl] = 0.5 * (1.0 - sgn[v])

                if last:
                    pltpu.sync_copy(dbufL, outs[chunk].at[pl.ds(row, B)])
                else:
                    pltpu.sync_copy(dbufL, outs[chunk].at[pl.ds(row, B)])
                    pltpu.sync_copy(dbufS, outs[NCHUNK + chunk].at[pl.ds(row, B)])

            plsc.subcore_barrier()

        for ci in range(2):
            @pl.when(core == ci)
            def _():
                for p in range(2):
                    run_pass(2 * ci + p)

    return up_kernel


def _make_down_phase(e_pad, n_acc, materialize):
    nb = e_pad // (NTILES * B)
    stripe = n_acc // NTILES
    n_in = 3 * NCHUNK if materialize else NCHUNK
    out_type = [jax.ShapeDtypeStruct((n_acc, CW), F32) for _ in range(NCHUNK)]

    @functools.partial(
        pl.kernel,
        mesh=_mesh(),
        out_type=out_type,
        compiler_params=pltpu.CompilerParams(use_tc_tiling_on_sc=False),
        scratch_types=[
            pltpu.VMEM((SB, B), I32),
            pltpu.VMEM((SB, B), I32),
            pltpu.VMEM((B, CW), F32),
            pltpu.VMEM((B, CW), F32),
            pltpu.VMEM((B, CW), F32),
            pltpu.VMEM_SHARED((n_acc, CW), F32),
            pltpu.SemaphoreType.DMA,
            pltpu.SemaphoreType.DMA,
            pltpu.SemaphoreType.DMA,
            pltpu.SemaphoreType.DMA,
        ],
    )
    def down_kernel(pidx, cidx, *refs):
        htabs = refs[:NCHUNK]
        if materialize:
            ltabs = refs[NCHUNK:2 * NCHUNK]
            stabs = refs[2 * NCHUNK:3 * NCHUNK]
        outs = refs[n_in:n_in + NCHUNK]
        (gsb, ssb, rowHa, rowHb, xbuf, accD,
         semHa, semHb, semHc, semHd) = refs[n_in + NCHUNK:]
        core = lax.axis_index("core")
        s = lax.axis_index("subcore")
        dbufD, dbufL, dbufS = rowHa, rowHb, xbuf

        def run_pass(chunk):
            _zero_fill(rowHb)
            _zero_stripe(rowHb, accD, stripe, s)
            plsc.subcore_barrier()
            _accumulate(pidx, cidx, nb, s, (htabs[chunk],), ((rowHa, rowHb),),
                        (accD,), gsb, ssb, ((semHa, semHb),), (semHc, semHd))
            plsc.subcore_barrier()

            @pl.loop(0, stripe, step=B)
            def _(r0):
                row = s * stripe + r0
                pltpu.sync_copy(accD.at[pl.ds(row, B)], dbufD)
                if materialize:
                    pltpu.sync_copy(ltabs[chunk].at[pl.ds(row, B)], dbufL)
                    pltpu.sync_copy(stabs[chunk].at[pl.ds(row, B)], dbufS)

                    @pl.loop(0, B)
                    def _(r):
                        sgn = _parity_to_sign(dbufS, r)
                        for v in range(CW // 16):
                            sl = pl.ds(16 * v, 16)
                            dbufD[r, sl] = (
                                sgn[v] * jnp.exp(dbufL[r, sl]) + dbufD[r, sl])

                pltpu.sync_copy(dbufD, outs[chunk].at[pl.ds(row, B)])

            plsc.subcore_barrier()

        for ci in range(2):
            @pl.when(core == ci)
            def _():
                for p in range(2):
                    run_pass(2 * ci + p)

    return down_kernel


def _tc_prep(feat):
    rb = 200
    grid = N2 // rb

    def body(feat_ref, *outs):
        x = feat_ref[...]
        logx = jnp.where(x == 0.0, F32(NEG_BIG), jnp.log(jnp.abs(x)))
        sgn = (x < 0.0).astype(F32)
        for c in range(NCHUNK):
            outs[c][...] = logx[:, 32 * c:32 * c + 32]
            outs[NCHUNK + c][...] = sgn[:, 32 * c:32 * c + 32]

    return pl.pallas_call(
        body,
        grid=(grid,),
        in_specs=[pl.BlockSpec((rb, 128), lambda i: (i, 0))],
        out_specs=[pl.BlockSpec((rb, CW), lambda i: (i, 0))] * (2 * NCHUNK),
        out_shape=[jax.ShapeDtypeStruct((N2, CW), F32)] * (2 * NCHUNK),
    )(feat)


def _tc_final(feat, d1c, W, b):
    rb = 400
    grid = N1 // rb

    def body(feat_ref, d0, d1, d2, d3, w_ref, b_ref, out_ref):
        d = jnp.concatenate([d0[...], d1[...], d2[...], d3[...]], axis=1)
        h = feat_ref[...] + d
        y = jax.lax.dot_general(h, w_ref[...], (((1,), (0,)), ((), ())),
                                preferred_element_type=F32,
                                precision=jax.lax.Precision.HIGHEST)
        y = y + b_ref[...]
        out_ref[...] = y * (1.0 / (1.0 + jnp.exp(-y)))

    return pl.pallas_call(
        body,
        grid=(grid,),
        in_specs=[pl.BlockSpec((rb, 128), lambda i: (i, 0))]
        + [pl.BlockSpec((rb, CW), lambda i: (i, 0))] * NCHUNK
        + [pl.BlockSpec((128, 128), lambda i: (0, 0)),
           pl.BlockSpec((128,), lambda i: (0,))],
        out_specs=pl.BlockSpec((rb, 128), lambda i: (i, 0)),
        out_shape=jax.ShapeDtypeStruct((N1, 128), F32),
    )(feat, *d1c, W, b)


def kernel(feat, edge_index_12, edge_index_23, edge_index_34, W, b):
    e12, e23, e34 = 507904, 106496, 24576

    c12u = _pad_to(edge_index_12[0], e12, 0)
    p12u = _pad_to(edge_index_12[1], e12, N2)
    c23u = _pad_to(edge_index_23[0], e23, 0)
    p23u = _pad_to(edge_index_23[1], e23, N3)
    c34u = _pad_to(edge_index_34[0], e34, 0)
    p34u = _pad_to(edge_index_34[1], e34, N4)
    p34d = _pad_to(edge_index_34[1], e34, 0)
    c34d = _pad_to(edge_index_34[0], e34, N3)
    p23d = _pad_to(edge_index_23[1], e23, 0)
    c23d = _pad_to(edge_index_23[0], e23, N2)
    p12d = _pad_to(edge_index_12[1], e12, 0)
    c12d = _pad_to(edge_index_12[0], e12, N1)

    tabs1 = _tc_prep(feat)

    up12 = _make_up_phase(e12, A2, last=False)
    tabs2 = up12(c12u, p12u, *tabs1)
    up23 = _make_up_phase(e23, A3, last=False)
    tabs3 = up23(c23u, p23u, *tabs2)
    up34 = _make_up_phase(e34, A4, last=True)
    h4 = up34(c34u, p34u, *tabs3)

    down34 = _make_down_phase(e34, A3, materialize=True)
    h3 = down34(p34d, c34d, *h4, *tabs3)
    down23 = _make_down_phase(e23, A2, materialize=True)
    h2 = down23(p23d, c23d, *h3, *tabs2)
    down12 = _make_down_phase(e12, A1, materialize=False)
    d1 = down12(p12d, c12d, *h2)

    return _tc_final(feat, d1, W, b)
